# baseline (speedup 1.0000x reference)
import functools

import jax
import jax.numpy as jnp
from jax import lax
from jax.experimental import pallas as pl
from jax.experimental.pallas import tpu as pltpu
from jax.experimental.pallas import tpu_sc as plsc

BATCH = 16384
D = 128
LANES = 16
NUM_WORKERS = 32
ROWS_PER_W = BATCH // NUM_WORKERS
CHUNK = 128
NCHUNK = ROWS_PER_W // CHUNK
NGROUP = CHUNK // LANES

_mesh = plsc.VectorSubcoreMesh(core_axis_name="c", subcore_axis_name="s")


@functools.partial(
    pl.kernel,
    mesh=_mesh,
    out_type=jax.ShapeDtypeStruct((BATCH,), jnp.float32),
    compiler_params=pltpu.CompilerParams(needs_layout_passes=False),
    scratch_types=[
        pltpu.VMEM((NCHUNK, CHUNK), jnp.int32),
        pltpu.VMEM((NCHUNK, CHUNK), jnp.int32),
        pltpu.VMEM((CHUNK, D), jnp.float32),
        pltpu.VMEM((CHUNK, D), jnp.float32),
        pltpu.VMEM((CHUNK, D), jnp.float32),
        pltpu.VMEM((CHUNK, D), jnp.float32),
        pltpu.VMEM((NGROUP * LANES * LANES,), jnp.float32),
        pltpu.VMEM((ROWS_PER_W,), jnp.float32),
        pltpu.SemaphoreType.DMA,
        pltpu.SemaphoreType.DMA,
        pltpu.SemaphoreType.DMA,
        pltpu.SemaphoreType.DMA,
    ],
)
def _mf_kernel(users, items, utab, itab, out,
               uidx, iidx, urows_a, irows_a, urows_b, irows_b, pbuf, outv,
               semu_a, semi_a, semu_b, semi_b):
    cid = lax.axis_index("c")
    sid = lax.axis_index("s")
    wid = sid * 2 + cid
    lanes16 = lax.iota(jnp.int32, LANES)

    pltpu.sync_copy(users.at[pl.ds(wid * NCHUNK, NCHUNK)], uidx)
    pltpu.sync_copy(items.at[pl.ds(wid * NCHUNK, NCHUNK)], iidx)

    bufs = [
        (urows_a, irows_a, semu_a, semi_a),
        (urows_b, irows_b, semu_b, semi_b),
    ]

    def start_gather(g):
        ur, ir, su, si = bufs[g % 2]
        cu = pltpu.async_copy(utab.at[uidx.at[g]], ur, su)
        ci = pltpu.async_copy(itab.at[iidx.at[g]], ir, si)
        return cu, ci

    idx_base = lanes16 * LANES

    def compute_chunk(g):
        ur, ir, _, _ = bufs[g % 2]

        @plsc.parallel_loop(0, NGROUP, unroll=2)
        def group_body(g2):
            row0 = g2 * LANES
            pb = g2 * (LANES * LANES)
            for r in range(LANES):
                row = row0 + r
                ms = [ur[row, pl.ds(d * LANES, LANES)]
                      * ir[row, pl.ds(d * LANES, LANES)]
                      for d in range(D // LANES)]
                while len(ms) > 1:
                    ms = [ms[i] + ms[i + 1] for i in range(0, len(ms), 2)]
                pbuf[pl.ds(pb + r * LANES, LANES)] = ms[0]
            gs = [plsc.load_gather(pbuf, [idx_base + pb + l])
                  for l in range(LANES)]
            while len(gs) > 1:
                gs = [gs[i] + gs[i + 1] for i in range(0, len(gs), 2)]
            sig = 1.0 / (1.0 + jnp.exp(-gs[0]))
            outv[pl.ds(g * CHUNK + row0, LANES)] = sig

    copies = {0: start_gather(0)}
    for g in range(NCHUNK):
        if g + 1 < NCHUNK:
            copies[g + 1] = start_gather(g + 1)
        cu, ci = copies[g]
        cu.wait()
        ci.wait()
        compute_chunk(g)

    pltpu.sync_copy(outv, out.at[pl.ds(wid * ROWS_PER_W, ROWS_PER_W)])


def kernel(users, items, embedding_user, embedding_item):
    users2d = users.reshape(NUM_WORKERS * NCHUNK, CHUNK)
    items2d = items.reshape(NUM_WORKERS * NCHUNK, CHUNK)
    return _mf_kernel(users2d, items2d, embedding_user, embedding_item)


# speedup vs baseline: 1.1548x; 1.1548x over previous
import functools

import jax
import jax.numpy as jnp
from jax import lax
from jax.experimental import pallas as pl
from jax.experimental.pallas import tpu as pltpu
from jax.experimental.pallas import tpu_sc as plsc

BATCH = 16384
D = 128
LANES = 16
NUM_WORKERS = 32
ROWS_PER_W = BATCH // NUM_WORKERS
CHUNK = 128
NCHUNK = ROWS_PER_W // CHUNK
NGROUP = CHUNK // LANES

_mesh = plsc.VectorSubcoreMesh(core_axis_name="c", subcore_axis_name="s")


@functools.partial(
    pl.kernel,
    mesh=_mesh,
    out_type=jax.ShapeDtypeStruct((BATCH,), jnp.float32),
    compiler_params=pltpu.CompilerParams(
        needs_layout_passes=False,
        disable_bounds_checks=True,
        disable_semaphore_checks=True,
    ),
    scratch_types=[
        pltpu.VMEM((NCHUNK, CHUNK), jnp.int32),
        pltpu.VMEM((NCHUNK, CHUNK), jnp.int32),
        pltpu.VMEM((2 * CHUNK, D), jnp.float32),
        pltpu.VMEM((2 * CHUNK, D), jnp.float32),
        pltpu.VMEM((NGROUP * LANES * LANES,), jnp.float32),
        pltpu.VMEM((ROWS_PER_W,), jnp.float32),
        pltpu.SemaphoreType.DMA,
        pltpu.SemaphoreType.DMA,
        pltpu.SemaphoreType.DMA,
        pltpu.SemaphoreType.DMA,
    ],
)
def _mf_kernel(users, items, utab, itab, out,
               uidx, iidx, urows, irows, pbuf, outv,
               semu_a, semi_a, semu_b, semi_b):
    cid = lax.axis_index("c")
    sid = lax.axis_index("s")
    wid = sid * 2 + cid
    lanes16 = lax.iota(jnp.int32, LANES)
    idx_base = lanes16 * LANES

    pltpu.sync_copy(users.at[pl.ds(wid * NCHUNK, NCHUNK)], uidx)
    pltpu.sync_copy(items.at[pl.ds(wid * NCHUNK, NCHUNK)], iidx)

    def start_gather(g, half):
        su, si = (semu_a, semi_a) if half == 0 else (semu_b, semi_b)
        r0 = half * CHUNK
        pltpu.async_copy(utab.at[uidx.at[g]], urows.at[pl.ds(r0, CHUNK)], su)
        pltpu.async_copy(itab.at[iidx.at[g]], irows.at[pl.ds(r0, CHUNK)], si)

    def wait_gather(half):
        su, si = (semu_a, semi_a) if half == 0 else (semu_b, semi_b)
        r0 = half * CHUNK
        pltpu.make_async_copy(
            utab.at[uidx.at[0]], urows.at[pl.ds(r0, CHUNK)], su).wait()
        pltpu.make_async_copy(
            itab.at[iidx.at[0]], irows.at[pl.ds(r0, CHUNK)], si).wait()

    start_gather(0, 0)

    def chunk_body(g, _):
        parity = lax.rem(g, 2)

        @pl.when(jnp.logical_and(parity == 0, g + 1 < NCHUNK))
        def _():
            start_gather(g + 1, 1)

        @pl.when(jnp.logical_and(parity == 1, g + 1 < NCHUNK))
        def _():
            start_gather(g + 1, 0)

        @pl.when(parity == 0)
        def _():
            wait_gather(0)

        @pl.when(parity == 1)
        def _():
            wait_gather(1)

        b0 = parity * CHUNK

        @plsc.parallel_loop(0, NGROUP, unroll=2)
        def group_body(g2):
            row0 = g2 * LANES
            pb = g2 * (LANES * LANES)
            for r in range(LANES):
                row = b0 + row0 + r
                ms = [urows[row, pl.ds(d * LANES, LANES)]
                      * irows[row, pl.ds(d * LANES, LANES)]
                      for d in range(D // LANES)]
                while len(ms) > 1:
                    ms = [ms[i] + ms[i + 1] for i in range(0, len(ms), 2)]
                pbuf[pl.ds(pb + r * LANES, LANES)] = ms[0]
            gs = [plsc.load_gather(pbuf, [idx_base + pb + l])
                  for l in range(LANES)]
            while len(gs) > 1:
                gs = [gs[i] + gs[i + 1] for i in range(0, len(gs), 2)]
            sig = 1.0 / (1.0 + jnp.exp(-gs[0]))
            outv[pl.ds(g * CHUNK + row0, LANES)] = sig

        return 0

    lax.fori_loop(0, NCHUNK, chunk_body, 0)
    pltpu.sync_copy(outv, out.at[pl.ds(wid * ROWS_PER_W, ROWS_PER_W)])


def kernel(users, items, embedding_user, embedding_item):
    users2d = users.reshape(NUM_WORKERS * NCHUNK, CHUNK)
    items2d = items.reshape(NUM_WORKERS * NCHUNK, CHUNK)
    return _mf_kernel(users2d, items2d, embedding_user, embedding_item)


# speedup vs baseline: 1.1818x; 1.0234x over previous
import functools

import jax
import jax.numpy as jnp
from jax import lax
from jax.experimental import pallas as pl
from jax.experimental.pallas import tpu as pltpu
from jax.experimental.pallas import tpu_sc as plsc

BATCH = 16384
D = 128
LANES = 16
NUM_WORKERS = 32
ROWS_PER_W = BATCH // NUM_WORKERS
CHUNK = 64
NCHUNK = ROWS_PER_W // CHUNK
NGROUP = CHUNK // LANES

_mesh = plsc.VectorSubcoreMesh(core_axis_name="c", subcore_axis_name="s")


@functools.partial(
    pl.kernel,
    mesh=_mesh,
    out_type=jax.ShapeDtypeStruct((BATCH,), jnp.float32),
    compiler_params=pltpu.CompilerParams(
        needs_layout_passes=False,
        disable_bounds_checks=True,
        disable_semaphore_checks=True,
    ),
    scratch_types=[
        pltpu.VMEM((NCHUNK, CHUNK), jnp.int32),
        pltpu.VMEM((NCHUNK, CHUNK), jnp.int32),
        pltpu.VMEM((2 * CHUNK, D), jnp.float32),
        pltpu.VMEM((2 * CHUNK, D), jnp.float32),
        pltpu.VMEM((NGROUP * LANES * LANES,), jnp.float32),
        pltpu.VMEM((ROWS_PER_W,), jnp.float32),
        pltpu.SemaphoreType.DMA,
        pltpu.SemaphoreType.DMA,
        pltpu.SemaphoreType.DMA,
        pltpu.SemaphoreType.DMA,
        pltpu.SemaphoreType.DMA,
        pltpu.SemaphoreType.DMA,
    ],
)
def _mf_kernel(users, items, utab, itab, out,
               uidx, iidx, urows, irows, pbuf, outv,
               semu_a, semi_a, semu_b, semi_b, semx, semy):
    cid = lax.axis_index("c")
    sid = lax.axis_index("s")
    wid = sid * 2 + cid
    lanes16 = lax.iota(jnp.int32, LANES)
    idx_base = lanes16 * LANES

    cx = pltpu.async_copy(users.at[pl.ds(wid * NCHUNK, NCHUNK)], uidx, semx)
    cy = pltpu.async_copy(items.at[pl.ds(wid * NCHUNK, NCHUNK)], iidx, semy)
    cx.wait()
    cy.wait()

    def start_gather(g, half):
        su, si = (semu_a, semi_a) if half == 0 else (semu_b, semi_b)
        r0 = half * CHUNK
        pltpu.async_copy(utab.at[uidx.at[g]], urows.at[pl.ds(r0, CHUNK)], su)
        pltpu.async_copy(itab.at[iidx.at[g]], irows.at[pl.ds(r0, CHUNK)], si)

    def wait_gather(half):
        su, si = (semu_a, semi_a) if half == 0 else (semu_b, semi_b)
        r0 = half * CHUNK
        pltpu.make_async_copy(
            utab.at[uidx.at[0]], urows.at[pl.ds(r0, CHUNK)], su).wait()
        pltpu.make_async_copy(
            itab.at[iidx.at[0]], irows.at[pl.ds(r0, CHUNK)], si).wait()

    start_gather(0, 0)

    def chunk_body(g, _):
        parity = lax.rem(g, 2)

        @pl.when(jnp.logical_and(parity == 0, g + 1 < NCHUNK))
        def _():
            start_gather(g + 1, 1)

        @pl.when(jnp.logical_and(parity == 1, g + 1 < NCHUNK))
        def _():
            start_gather(g + 1, 0)

        @pl.when(parity == 0)
        def _():
            wait_gather(0)

        @pl.when(parity == 1)
        def _():
            wait_gather(1)

        b0 = parity * CHUNK

        @plsc.parallel_loop(0, NGROUP, unroll=2)
        def group_body(g2):
            row0 = g2 * LANES
            pb = g2 * (LANES * LANES)
            for r in range(LANES):
                row = b0 + row0 + r
                ms = [urows[row, pl.ds(d * LANES, LANES)]
                      * irows[row, pl.ds(d * LANES, LANES)]
                      for d in range(D // LANES)]
                while len(ms) > 1:
                    ms = [ms[i] + ms[i + 1] for i in range(0, len(ms), 2)]
                pbuf[pl.ds(pb + r * LANES, LANES)] = ms[0]
            gs = [plsc.load_gather(pbuf, [idx_base + pb + l])
                  for l in range(LANES)]
            while len(gs) > 1:
                gs = [gs[i] + gs[i + 1] for i in range(0, len(gs), 2)]
            sig = 1.0 / (1.0 + jnp.exp(-gs[0]))
            outv[pl.ds(g * CHUNK + row0, LANES)] = sig

        return 0

    lax.fori_loop(0, NCHUNK, chunk_body, 0)
    pltpu.sync_copy(outv, out.at[pl.ds(wid * ROWS_PER_W, ROWS_PER_W)])


def kernel(users, items, embedding_user, embedding_item):
    users2d = users.reshape(NUM_WORKERS * NCHUNK, CHUNK)
    items2d = items.reshape(NUM_WORKERS * NCHUNK, CHUNK)
    return _mf_kernel(users2d, items2d, embedding_user, embedding_item)
